# Initial kernel scaffold; baseline (speedup 1.0000x reference)
import functools

import jax
import jax.numpy as jnp
from jax import lax
from jax.experimental import pallas as pl
from jax.experimental.pallas import tpu as pltpu
from jax.experimental.pallas import tpu_sc as plsc

B = 4096
SEQ = 200
D = 32
LANES = 16
NC = 2
NS = 16
NW = NC * NS
RPW = B // NW
CH1 = 104
CH2 = 96


def _body(text_hbm, lens_hbm, table_hbm, out_hbm,
          text_v, lens_v, rows0, rows1, out_v, sem0, sem1):
    wid = lax.axis_index("s") * NC + lax.axis_index("c")
    base = wid * RPW

    pltpu.sync_copy(text_hbm.at[pl.ds(base, RPW), :], text_v)
    pltpu.sync_copy(lens_hbm.at[pl.ds(base, RPW)], lens_v)

    def fire(r, rows_v, sem):
        idx1 = text_v.at[r, pl.ds(0, CH1)]
        idx2 = text_v.at[r, pl.ds(CH1, CH2)]
        pltpu.async_copy(table_hbm.at[idx1], rows_v.at[pl.ds(0, CH1), :], sem)
        pltpu.async_copy(table_hbm.at[idx2], rows_v.at[pl.ds(CH1, CH2), :], sem)

    def wait(r, rows_v, sem):
        idx1 = text_v.at[r, pl.ds(0, CH1)]
        idx2 = text_v.at[r, pl.ds(CH1, CH2)]
        pltpu.make_async_copy(table_hbm.at[idx1],
                              rows_v.at[pl.ds(0, CH1), :], sem).wait()
        pltpu.make_async_copy(table_hbm.at[idx2],
                              rows_v.at[pl.ds(CH1, CH2), :], sem).wait()

    def accumulate(r, rows_v):
        len_vec = plsc.load_gather(lens_v, [jnp.broadcast_to(r, (LANES,))])
        len_s = jnp.max(len_vec)
        n8 = len_s // 8

        def chunk_body(c, carry):
            a0, a1 = carry
            t0 = c * 8
            for u in range(8):
                a0 = a0 + rows_v[t0 + u, 0:16]
                a1 = a1 + rows_v[t0 + u, 16:32]
            return a0, a1

        zero = jnp.zeros((LANES,), jnp.float32)
        acc0, acc1 = lax.fori_loop(0, n8, chunk_body, (zero, zero))

        def rem_body(t, carry):
            a0, a1 = carry
            return a0 + rows_v[t, 0:16], a1 + rows_v[t, 16:32]

        acc0, acc1 = lax.fori_loop(n8 * 8, len_s, rem_body, (acc0, acc1))

        inv = 1.0 / len_vec.astype(jnp.float32)
        out_v[r, 0:16] = acc0 * inv
        out_v[r, 16:32] = acc1 * inv

    fire(0, rows0, sem0)

    def outer(i, _):
        r0 = 2 * i
        r1 = 2 * i + 1
        fire(r1, rows1, sem1)
        wait(r0, rows0, sem0)
        accumulate(r0, rows0)

        @pl.when(i < RPW // 2 - 1)
        def _():
            fire(r0 + 2, rows0, sem0)

        wait(r1, rows1, sem1)
        accumulate(r1, rows1)
        return 0

    lax.fori_loop(0, RPW // 2, outer, 0)

    pltpu.sync_copy(out_v, out_hbm.at[pl.ds(base, RPW), :])


@functools.partial(
    pl.kernel,
    out_type=jax.ShapeDtypeStruct((B, D), jnp.float32),
    mesh=plsc.VectorSubcoreMesh(core_axis_name="c", subcore_axis_name="s"),
    scratch_types=[
        pltpu.VMEM((RPW, SEQ), jnp.int32),
        pltpu.VMEM((RPW,), jnp.int32),
        pltpu.VMEM((SEQ, D), jnp.float32),
        pltpu.VMEM((SEQ, D), jnp.float32),
        pltpu.VMEM((RPW, D), jnp.float32),
        pltpu.SemaphoreType.DMA,
        pltpu.SemaphoreType.DMA,
    ],
)
def _encode(text_hbm, lens_hbm, table_hbm, out_hbm,
            text_v, lens_v, rows0, rows1, out_v, sem0, sem1):
    _body(text_hbm, lens_hbm, table_hbm, out_hbm,
          text_v, lens_v, rows0, rows1, out_v, sem0, sem1)


def kernel(text, text_len, emb_weight):
    return _encode(text.astype(jnp.int32), text_len, emb_weight)


# speedup vs baseline: 2.3233x; 2.3233x over previous
import functools

import jax
import jax.numpy as jnp
from jax import lax
from jax.experimental import pallas as pl
from jax.experimental.pallas import tpu as pltpu
from jax.experimental.pallas import tpu_sc as plsc

B = 4096
SEQ = 200
D = 32
LANES = 16
NC = 2
NS = 16
NW = NC * NS
RPW = B // NW
CH1 = 104
CH2 = 96


def _body(text_hbm, lens_hbm, table_hbm, out_hbm,
          text_v, lens_v, rows0, rows1, out_v, sem0, sem1):
    wid = lax.axis_index("s") * NC + lax.axis_index("c")
    base = wid * RPW

    pltpu.sync_copy(text_hbm.at[pl.ds(base, RPW), :], text_v)
    pltpu.sync_copy(lens_hbm.at[pl.ds(base, RPW)], lens_v)

    def fire(r, rows_v, sem):
        idx1 = text_v.at[r, pl.ds(0, CH1)]
        idx2 = text_v.at[r, pl.ds(CH1, CH2)]
        pltpu.async_copy(table_hbm.at[idx1], rows_v.at[pl.ds(0, CH1), :], sem)
        pltpu.async_copy(table_hbm.at[idx2], rows_v.at[pl.ds(CH1, CH2), :], sem)

    def wait(r, rows_v, sem):
        idx1 = text_v.at[r, pl.ds(0, CH1)]
        idx2 = text_v.at[r, pl.ds(CH1, CH2)]
        pltpu.make_async_copy(table_hbm.at[idx1],
                              rows_v.at[pl.ds(0, CH1), :], sem).wait()
        pltpu.make_async_copy(table_hbm.at[idx2],
                              rows_v.at[pl.ds(CH1, CH2), :], sem).wait()

    def accumulate(r, rows_v):
        len_vec = plsc.load_gather(lens_v, [jnp.broadcast_to(r, (LANES,))])
        len_s = jnp.max(len_vec)
        n8 = len_s // 8

        def chunk_body(c, carry):
            a0, a1 = carry
            t0 = c * 8
            for u in range(8):
                a0 = a0 + rows_v[t0 + u, 0:16]
                a1 = a1 + rows_v[t0 + u, 16:32]
            return a0, a1

        zero = jnp.zeros((LANES,), jnp.float32)
        acc0, acc1 = lax.fori_loop(0, n8, chunk_body, (zero, zero))

        def rem_body(t, carry):
            a0, a1 = carry
            return a0 + rows_v[t, 0:16], a1 + rows_v[t, 16:32]

        acc0, acc1 = lax.fori_loop(n8 * 8, len_s, rem_body, (acc0, acc1))

        inv = 1.0 / len_vec.astype(jnp.float32)
        out_v[r, 0:16] = acc0 * inv
        out_v[r, 16:32] = acc1 * inv

    fire(0, rows0, sem0)

    def outer(i, _):
        r0 = 2 * i
        r1 = 2 * i + 1
        fire(r1, rows1, sem1)
        wait(r0, rows0, sem0)
        accumulate(r0, rows0)

        @pl.when(i < RPW // 2 - 1)
        def _():
            fire(r0 + 2, rows0, sem0)

        wait(r1, rows1, sem1)
        accumulate(r1, rows1)
        return 0

    lax.fori_loop(0, RPW // 2, outer, 0)

    pltpu.sync_copy(out_v, out_hbm.at[pl.ds(base, RPW), :])


@functools.partial(
    pl.kernel,
    out_type=jax.ShapeDtypeStruct((B, D), jnp.float32),
    mesh=plsc.VectorSubcoreMesh(core_axis_name="c", subcore_axis_name="s"),
    compiler_params=pltpu.CompilerParams(
        use_tc_tiling_on_sc=False, needs_layout_passes=False),
    scratch_types=[
        pltpu.VMEM((RPW, SEQ), jnp.int32),
        pltpu.VMEM((RPW,), jnp.int32),
        pltpu.VMEM((SEQ, D), jnp.float32),
        pltpu.VMEM((SEQ, D), jnp.float32),
        pltpu.VMEM((RPW, D), jnp.float32),
        pltpu.SemaphoreType.DMA,
        pltpu.SemaphoreType.DMA,
    ],
)
def _encode(text_hbm, lens_hbm, table_hbm, out_hbm,
            text_v, lens_v, rows0, rows1, out_v, sem0, sem1):
    _body(text_hbm, lens_hbm, table_hbm, out_hbm,
          text_v, lens_v, rows0, rows1, out_v, sem0, sem1)


def kernel(text, text_len, emb_weight):
    return _encode(text.astype(jnp.int32), text_len, emb_weight)
